# baseline (speedup 1.0000x reference)
import functools
import math

import jax
import jax.numpy as jnp
from jax import lax
from jax.experimental import pallas as pl
from jax.experimental.pallas import tpu as pltpu
from jax.experimental.pallas import tpu_sc as plsc

EMB = 64
SCALE = math.sqrt(EMB)

_info = plsc.get_sparse_core_info()
NC = _info.num_cores
NS = _info.num_subcores
L = _info.num_lanes
NW = NC * NS

NB = 4096
NT = 200
LANES = 128
SUB = 8
NBT = NB // LANES

_mesh = plsc.VectorSubcoreMesh(core_axis_name="c", subcore_axis_name="s")


@functools.partial(
    pl.kernel,
    out_type=jax.ShapeDtypeStruct((NT, SUB, NBT, SUB, LANES), jnp.float32),
    mesh=_mesh,
    compiler_params=pltpu.CompilerParams(
        use_tc_tiling_on_sc=False, needs_layout_passes=False),
    scratch_types=[
        pltpu.VMEM((LANES, NT), jnp.int32),
        pltpu.VMEM((NT, LANES), jnp.int32),
        pltpu.VMEM((LANES, EMB), jnp.float32),
        pltpu.VMEM((LANES, EMB), jnp.float32),
        pltpu.VMEM((SUB, SUB, LANES), jnp.float32),
        pltpu.VMEM((SUB, SUB, LANES), jnp.float32),
        pltpu.SemaphoreType.DMA,
        pltpu.SemaphoreType.DMA,
        pltpu.SemaphoreType.DMA,
        pltpu.SemaphoreType.DMA,
    ],
)
def _emb_kernel(tokens_hbm, table_hbm, out_hbm, tok_raw, idx_v, rows0, rows1,
                ob0, ob1, gsem0, gsem1, ssem0, ssem1):
    rows = (rows0, rows1)
    ob = (ob0, ob1)
    gsem = (gsem0, gsem1)
    ssem = (ssem0, ssem1)

    w = lax.axis_index("s") * NC + lax.axis_index("c")

    pltpu.sync_copy(tokens_hbm.at[pl.ds(w * LANES, LANES)], tok_raw)

    iota = lax.iota(jnp.int32, L)
    zeros = jnp.zeros((L,), jnp.int32)

    @plsc.parallel_loop(0, NT * (LANES // L), unroll=8)
    def _(q):
        t = q // (LANES // L)
        l = q % (LANES // L)
        v = plsc.load_gather(tok_raw, [iota + l * L, zeros + t])
        idx_v[t, pl.ds(l * L, L)] = v

    def gather_start(t, b):
        pltpu.async_copy(table_hbm.at[idx_v.at[t]], rows[b], gsem[b])

    def gather_wait(t, b):
        pltpu.make_async_copy(
            table_hbm.at[idx_v.at[t]], rows[b], gsem[b]).wait()

    def store_start(t, b):
        pltpu.async_copy(ob[b], out_hbm.at[t, pl.ds(0, SUB), w], ssem[b])

    def store_wait(t, b):
        pltpu.make_async_copy(
            ob[b], out_hbm.at[t, pl.ds(0, SUB), w], ssem[b]).wait()

    idx_ct = tuple((iota + j * L) // SUB for j in range(EMB // L))
    idx_cs = tuple((iota + j * L) % SUB for j in range(EMB // L))

    def transpose_scale(b):
        @plsc.parallel_loop(0, LANES, unroll=8)
        def _(r):
            idx_r = zeros + r
            for j in range(EMB // L):
                v = rows[b][r, pl.ds(j * L, L)]
                plsc.store_scatter(
                    ob[b], [idx_ct[j], idx_cs[j], idx_r], v * SCALE)

    gather_start(0, 0)
    gather_start(1, 1)

    def body(g, _):
        for b in range(2):
            t = g * 2 + b

            gather_wait(t, b)

            @pl.when(t >= 2)
            def _():
                store_wait(t - 2, b)

            transpose_scale(b)
            store_start(t, b)

            @pl.when(t + 2 < NT)
            def _():
                gather_start(t + 2, b)

        return 0

    lax.fori_loop(0, NT // 2, body, 0)
    store_wait(NT - 2, 0)
    store_wait(NT - 1, 1)


def kernel(tokens, table):
    out5 = _emb_kernel(tokens.astype(jnp.int32), table)
    return out5.transpose(2, 4, 0, 1, 3).reshape(NB, NT, EMB)


# speedup vs baseline: 1.6640x; 1.6640x over previous
import functools
import math

import jax
import jax.numpy as jnp
from jax import lax
from jax.experimental import pallas as pl
from jax.experimental.pallas import tpu as pltpu
from jax.experimental.pallas import tpu_sc as plsc

EMB = 64
SCALE = math.sqrt(EMB)

_info = plsc.get_sparse_core_info()
NC = _info.num_cores
NS = _info.num_subcores
L = _info.num_lanes
NW = NC * NS

NB = 4096
NT = 200
LANES = 128
SUB = 8
NBT = NB // LANES
OBP = 137

_mesh = plsc.VectorSubcoreMesh(core_axis_name="c", subcore_axis_name="s")


@functools.partial(
    pl.kernel,
    out_type=jax.ShapeDtypeStruct((NT, SUB, NBT, SUB, LANES), jnp.float32),
    mesh=_mesh,
    compiler_params=pltpu.CompilerParams(
        use_tc_tiling_on_sc=False, needs_layout_passes=False),
    scratch_types=[
        pltpu.VMEM((NT, LANES), jnp.int32),
        pltpu.VMEM((LANES, EMB), jnp.float32),
        pltpu.VMEM((LANES, EMB), jnp.float32),
        pltpu.VMEM((SUB, SUB, OBP), jnp.float32),
        pltpu.VMEM((SUB, SUB, OBP), jnp.float32),
        pltpu.SemaphoreType.DMA,
        pltpu.SemaphoreType.DMA,
        pltpu.SemaphoreType.DMA,
        pltpu.SemaphoreType.DMA,
    ],
)
def _emb_kernel(tokens_hbm, table_hbm, out_hbm, idx_v, rows0, rows1,
                ob0, ob1, gsem0, gsem1, ssem0, ssem1):
    rows = (rows0, rows1)
    ob = (ob0, ob1)
    gsem = (gsem0, gsem1)
    ssem = (ssem0, ssem1)

    w = lax.axis_index("s") * NC + lax.axis_index("c")

    pltpu.sync_copy(tokens_hbm.at[pl.ds(0, NT), pl.ds(w * LANES, LANES)],
                    idx_v)

    def gather_start(t, b):
        pltpu.async_copy(table_hbm.at[idx_v.at[t]], rows[b], gsem[b])

    def gather_wait(t, b):
        pltpu.make_async_copy(
            table_hbm.at[idx_v.at[t]], rows[b], gsem[b]).wait()

    def out_slice(t):
        return out_hbm.at[t, pl.ds(0, SUB), w]

    def ob_slice(b):
        return ob[b].at[pl.ds(0, SUB), pl.ds(0, SUB), pl.ds(0, LANES)]

    def store_start(t, b):
        pltpu.async_copy(ob_slice(b), out_slice(t), ssem[b])

    def store_wait(t, b):
        pltpu.make_async_copy(ob_slice(b), out_slice(t), ssem[b]).wait()

    iota = lax.iota(jnp.int32, L)
    zeros = jnp.zeros((L,), jnp.int32)
    idx_ct = tuple((iota + j * L) // SUB for j in range(EMB // L))
    idx_cs = tuple((iota + j * L) % SUB for j in range(EMB // L))

    def transpose_scale(b):
        @plsc.parallel_loop(0, LANES, unroll=8)
        def _(r):
            idx_r = zeros + r
            for j in range(EMB // L):
                v = rows[b][r, pl.ds(j * L, L)]
                plsc.store_scatter(
                    ob[b], [idx_ct[j], idx_cs[j], idx_r], v * SCALE)

    gather_start(0, 0)
    gather_start(1, 1)

    def body(g, _):
        for b in range(2):
            t = g * 2 + b

            gather_wait(t, b)

            @pl.when(t >= 2)
            def _():
                store_wait(t - 2, b)

            transpose_scale(b)
            store_start(t, b)

            @pl.when(t + 2 < NT)
            def _():
                gather_start(t + 2, b)

        return 0

    lax.fori_loop(0, NT // 2, body, 0)
    store_wait(NT - 2, 0)
    store_wait(NT - 1, 1)


def kernel(tokens, table):
    out5 = _emb_kernel(tokens.T.astype(jnp.int32), table)
    return out5.transpose(2, 4, 0, 1, 3).reshape(NB, NT, EMB)
